# initial kernel scaffold (unmeasured)
import jax
import jax.numpy as jnp
from jax import lax
from jax.experimental import pallas as pl
from jax.experimental.pallas import tpu as pltpu

Bb, S, D, N = 8, 512, 512, 16


def kernel(x, A, B, C):
    dAT = jnp.exp(A).T

    def body(x_ref, dAT_ref, B_ref, C_ref, out_ref, h_ref,
             send_sem, recv_sem, ack_sem):
        my_x = lax.axis_index("x")
        my_y = lax.axis_index("y")

        rdma = pltpu.make_async_remote_copy(
            src_ref=h_ref,
            dst_ref=h_ref,
            send_sem=send_sem,
            recv_sem=recv_sem,
            device_id=(my_x, 1),
            device_id_type=pl.DeviceIdType.MESH,
        )

        @pl.when(my_y == 0)
        def _():
            h_ref[...] = jnp.zeros_like(h_ref)

        @pl.when(my_y == 1)
        def _():
            rdma.wait_recv()
            pl.semaphore_signal(
                ack_sem, inc=1,
                device_id=(my_x, 0),
                device_id_type=pl.DeviceIdType.MESH,
            )

        def step(t, carry):
            x_t = x_ref[:, t, :]
            y_t = jnp.zeros_like(x_t)
            for n in range(N):
                dA_n = dAT_ref[n:n + 1, :]
                b_tn = B_ref[:, t, n:n + 1]
                c_tn = C_ref[:, t, n:n + 1]
                h_n = h_ref[n] * dA_n + x_t * b_tn
                h_ref[n] = h_n
                y_t = y_t + h_n * c_tn
            out_ref[:, t, :] = y_t
            return carry

        lax.fori_loop(0, S, step, 0)

        @pl.when(my_y == 0)
        def _():
            rdma.start()
            rdma.wait_send()
            pl.semaphore_wait(ack_sem, 1)

    return pl.pallas_call(
        body,
        out_shape=jax.ShapeDtypeStruct((Bb, S, D), jnp.float32),
        in_specs=[pl.BlockSpec(memory_space=pltpu.VMEM)] * 4,
        out_specs=pl.BlockSpec(memory_space=pltpu.VMEM),
        scratch_shapes=[
            pltpu.VMEM((N, Bb, D), jnp.float32),
            pltpu.SemaphoreType.DMA,
            pltpu.SemaphoreType.DMA,
            pltpu.SemaphoreType.REGULAR,
        ],
        compiler_params=pltpu.CompilerParams(collective_id=0),
    )(x, dAT, B, C)


# baseline (device time: 403369 ns/iter reference)
import jax
import jax.numpy as jnp
from jax import lax
from jax.experimental import pallas as pl
from jax.experimental.pallas import tpu as pltpu

Bb, S, D, N = 8, 512, 512, 16


def kernel(x, A, B, C):
    dAT = jnp.exp(A).T

    def body(x_ref, dAT_ref, B_ref, C_ref, out_ref, h_ref,
             send_sem, recv_sem, ack_sem):
        my_x = lax.axis_index("x")
        my_y = lax.axis_index("y")

        rdma = pltpu.make_async_remote_copy(
            src_ref=h_ref,
            dst_ref=h_ref,
            send_sem=send_sem,
            recv_sem=recv_sem,
            device_id=(my_x, 1),
            device_id_type=pl.DeviceIdType.MESH,
        )

        @pl.when(my_y == 0)
        def _():
            h_ref[...] = jnp.zeros_like(h_ref)

        @pl.when(my_y == 1)
        def _():
            rdma.wait_recv()
            pl.semaphore_signal(
                ack_sem, inc=1,
                device_id=(my_x, 0),
                device_id_type=pl.DeviceIdType.MESH,
            )

        def step(t, carry):
            x_t = x_ref[:, t, :]
            y_t = jnp.zeros_like(x_t)
            for n in range(N):
                dA_n = dAT_ref[n:n + 1, :]
                b_tn = B_ref[:, t, n:n + 1]
                c_tn = C_ref[:, t, n:n + 1]
                h_n = h_ref[n] * dA_n + x_t * b_tn
                h_ref[n] = h_n
                y_t = y_t + h_n * c_tn
            out_ref[:, t, :] = y_t
            return carry

        lax.fori_loop(0, S, step, 0)

        @pl.when(my_y == 0)
        def _():
            rdma.start()
            rdma.wait_send()
            pl.semaphore_wait(ack_sem, 1)

    return pl.pallas_call(
        body,
        out_shape=jax.ShapeDtypeStruct((Bb, S, D), jnp.float32),
        in_specs=[pl.BlockSpec(memory_space=pltpu.VMEM)] * 4,
        out_specs=pl.BlockSpec(memory_space=pltpu.VMEM),
        scratch_shapes=[
            pltpu.VMEM((N, Bb, D), jnp.float32),
            pltpu.SemaphoreType.DMA,
            pltpu.SemaphoreType.DMA,
            pltpu.SemaphoreType.REGULAR,
        ],
    )(x, dAT, B, C)
